# baseline (speedup 1.0000x reference)
import functools

import jax
import jax.numpy as jnp
from jax.experimental import pallas as pl
from jax.experimental.pallas import tpu as pltpu

_N = 4096
_H = 256
_HE = 272
_BJP = 512
_BJA = 512
_NJA = _N // _BJA
_PHIGH = jax.lax.Precision.HIGHEST


def _pre0_body(m_ref, x_ref, w1_ref, b1c_ref, ws_ref, bsc_ref,
               inc_ref, mbf_ref, txt_ref, rest_ref):
    j = pl.program_id(0)

    @pl.when(j == 0)
    def _():
        inc_ref[...] = jnp.zeros_like(inc_ref)

    m = m_ref[...]
    inc_ref[...] += jnp.sum(m, axis=0, keepdims=True)
    mbf_ref[...] = m.astype(jnp.bfloat16)
    x = x_ref[...]
    txt = jax.lax.dot_general(
        w1_ref[...], x, (((0,), (1,)), ((), ())),
        preferred_element_type=jnp.float32,
        precision=_PHIGH) + b1c_ref[...]
    txt_ref[...] = jnp.concatenate(
        [txt.astype(jnp.bfloat16),
         jnp.ones((_HE - _H, _BJP), jnp.bfloat16)], axis=0)
    rest = jax.lax.dot_general(
        ws_ref[...], x, (((0,), (1,)), ((), ())),
        preferred_element_type=jnp.float32,
        precision=_PHIGH) + bsc_ref[...]
    rest_ref[...] = rest.astype(jnp.bfloat16)


def _aggepi_body(*args, has_next, apply_relu):
    if has_next:
        (mbf_ref, txts_ref, inc_ref, txtf_ref, rest_ref, w2_ref, b2c_ref,
         gc_ref, btc_ref, nw1_ref, nb1c_ref, ht_ref, ntxt_ref,
         acc_ref) = args
    else:
        (mbf_ref, txts_ref, inc_ref, txtf_ref, rest_ref, w2_ref, b2c_ref,
         gc_ref, btc_ref, out_ref, acc_ref) = args

    j = pl.program_id(0)
    inc_row = inc_ref[...]

    @pl.when(j < _NJA)
    def _():
        thr = (1e-6 * (inc_row + 1e-8)).astype(jnp.bfloat16)
        m = mbf_ref[...]
        mm = jnp.where(m > thr, m, jnp.zeros_like(m))
        part = jax.lax.dot_general(
            txts_ref[...], mm,
            (((1,), (0,)), ((), ())),
            preferred_element_type=jnp.float32)

        @pl.when(j == 0)
        def _():
            acc_ref[...] = part

        @pl.when(j > 0)
        def _():
            acc_ref[...] += part

    @pl.when(j == _NJA)
    def _():
        s_row = acc_ref[_H:_H + 1, :]
        denom_row = s_row + 1e-8 * (inc_row + 1e-8)
        txt = txtf_ref[0:_H, :].astype(jnp.float32)
        aggt = jnp.where(s_row > 0.0, acc_ref[0:_H, :] / denom_row, txt)
        outt = jax.lax.dot_general(
            w2_ref[...], aggt.astype(jnp.bfloat16), (((0,), (0,)), ((), ())),
            preferred_element_type=jnp.float32) + b2c_ref[...]
        outt = outt + rest_ref[...].astype(jnp.float32)
        mu = jnp.mean(outt, axis=0, keepdims=True)
        var = jnp.mean((outt - mu) ** 2, axis=0, keepdims=True)
        outt = ((outt - mu) * jax.lax.rsqrt(var + 1e-5) * gc_ref[...]
                + btc_ref[...])
        if apply_relu:
            outt = jnp.maximum(outt, 0.0)
        if has_next:
            ht_ref[...] = outt.astype(jnp.bfloat16)
            ntxt = jax.lax.dot_general(
                nw1_ref[...], outt.astype(jnp.bfloat16),
                (((0,), (0,)), ((), ())),
                preferred_element_type=jnp.float32) + nb1c_ref[...]
            ntxt_ref[...] = jnp.concatenate(
                [ntxt.astype(jnp.bfloat16),
                 jnp.ones((_HE - _H, _N), jnp.bfloat16)], axis=0)
        else:
            out_ref[...] = outt.T


def _aggepi(Mbf, txt_bf, inc, resT_bf, W2bf, b2, g, bt, next_w1bf=None,
            next_b1=None, apply_relu=False):
    col = lambda v: v.reshape(-1, 1)
    has_next = next_w1bf is not None
    last = _NJA - 1
    in_specs = [
        pl.BlockSpec((_BJA, _N),
                     lambda j: (jnp.minimum(j, last), 0)),
        pl.BlockSpec((_HE, _BJA),
                     lambda j: (0, jnp.minimum(j, last))),
        pl.BlockSpec((1, _N), lambda j: (0, 0)),
        pl.BlockSpec((_HE, _N), lambda j: (0, 0)),
        pl.BlockSpec((_H, _N), lambda j: (0, 0)),
        pl.BlockSpec((_H, _H), lambda j: (0, 0)),
        pl.BlockSpec((_H, 1), lambda j: (0, 0)),
        pl.BlockSpec((_H, 1), lambda j: (0, 0)),
        pl.BlockSpec((_H, 1), lambda j: (0, 0)),
    ]
    inputs = [Mbf, txt_bf, inc, txt_bf, resT_bf, W2bf, col(b2), col(g),
              col(bt)]
    if has_next:
        in_specs += [
            pl.BlockSpec((_H, _H), lambda j: (0, 0)),
            pl.BlockSpec((_H, 1), lambda j: (0, 0)),
        ]
        inputs += [next_w1bf, col(next_b1)]
        out_specs = [
            pl.BlockSpec((_H, _N), lambda j: (0, 0)),
            pl.BlockSpec((_HE, _N), lambda j: (0, 0)),
        ]
        out_shape = [
            jax.ShapeDtypeStruct((_H, _N), jnp.bfloat16),
            jax.ShapeDtypeStruct((_HE, _N), jnp.bfloat16),
        ]
    else:
        out_specs = pl.BlockSpec((_N, _H), lambda j: (0, 0))
        out_shape = jax.ShapeDtypeStruct((_N, _H), jnp.float32)

    body = functools.partial(_aggepi_body, has_next=has_next,
                             apply_relu=apply_relu)
    return pl.pallas_call(
        body,
        grid=(_NJA + 1,),
        in_specs=in_specs,
        out_specs=out_specs,
        out_shape=out_shape,
        scratch_shapes=[pltpu.VMEM((_HE, _N), jnp.float32)],
        compiler_params=pltpu.CompilerParams(
            dimension_semantics=("arbitrary",),
        ),
    )(*inputs)


def kernel(node_features, mobility_matrix, W1_0, b1_0, W2_0, b2_0, Ws_0,
           bs_0, g_0, bt_0, W1_1, b1_1, W2_1, b2_1, g_1, bt_1):
    col = lambda v: v.reshape(-1, 1)
    nJ = _N // _BJP
    inc, mbf, tx0t, res0t = pl.pallas_call(
        _pre0_body,
        grid=(nJ,),
        in_specs=[
            pl.BlockSpec((_BJP, _N), lambda j: (j, 0)),
            pl.BlockSpec((_BJP, 128), lambda j: (j, 0)),
            pl.BlockSpec((128, _H), lambda j: (0, 0)),
            pl.BlockSpec((_H, 1), lambda j: (0, 0)),
            pl.BlockSpec((128, _H), lambda j: (0, 0)),
            pl.BlockSpec((_H, 1), lambda j: (0, 0)),
        ],
        out_specs=[
            pl.BlockSpec((1, _N), lambda j: (0, 0)),
            pl.BlockSpec((_BJP, _N), lambda j: (j, 0)),
            pl.BlockSpec((_HE, _BJP), lambda j: (0, j)),
            pl.BlockSpec((_H, _BJP), lambda j: (0, j)),
        ],
        out_shape=[
            jax.ShapeDtypeStruct((1, _N), jnp.float32),
            jax.ShapeDtypeStruct((_N, _N), jnp.bfloat16),
            jax.ShapeDtypeStruct((_HE, _N), jnp.bfloat16),
            jax.ShapeDtypeStruct((_H, _N), jnp.bfloat16),
        ],
        compiler_params=pltpu.CompilerParams(
            dimension_semantics=("arbitrary",),
        ),
    )(mobility_matrix, node_features, W1_0, col(b1_0), Ws_0, col(bs_0))

    w2_0bf = W2_0.astype(jnp.bfloat16)
    w1_1bf = W1_1.astype(jnp.bfloat16)
    w2_1bf = W2_1.astype(jnp.bfloat16)

    ht, tx1t = _aggepi(mbf, tx0t, inc, res0t, w2_0bf, b2_0, g_0, bt_0,
                       next_w1bf=w1_1bf, next_b1=b1_1, apply_relu=False)
    out = _aggepi(mbf, tx1t, inc, ht, w2_1bf, b2_1, g_1, bt_1,
                  apply_relu=True)
    return out


# speedup vs baseline: 1.0314x; 1.0314x over previous
import jax
import jax.numpy as jnp
from jax.experimental import pallas as pl
from jax.experimental.pallas import tpu as pltpu

_N = 4096
_H = 256
_HE = 272
_BJP = 128
_BJA = 512
_NJP = _N // _BJP
_NJA = _N // _BJA
_EC = 1024
_PHIGH = jax.lax.Precision.HIGHEST

_J_AGG0 = _NJP
_J_EPI0 = _NJP + _NJA
_J_AGG1 = _J_EPI0 + 1
_J_EPI1 = _J_AGG1 + _NJA


def _body(m_ref, x_ref, w1_ref, b1r_ref, ws_ref, bsr_ref,
          w20_ref, b20c_ref, g0c_ref, bt0c_ref,
          w11_ref, b11r_ref, w21_ref, b21c_ref, g1c_ref, bt1c_ref,
          out_ref,
          mbf_s, txt_s, res_s, ht_s, acc_s, inc_s):
    j = pl.program_id(0)

    @pl.when(j < _NJP)
    def _():
        @pl.when(j == 0)
        def _():
            inc_s[...] = jnp.zeros_like(inc_s)

        m = m_ref[...]
        inc_s[...] += jnp.sum(m, axis=0, keepdims=True)
        mbf_s[pl.ds(j * _BJP, _BJP), :] = m.astype(jnp.bfloat16)
        x = x_ref[...]
        tx = jax.lax.dot_general(
            x, w1_ref[...], (((1,), (0,)), ((), ())),
            preferred_element_type=jnp.float32,
            precision=_PHIGH) + b1r_ref[...]
        txt_s[pl.ds(j * _BJP, _BJP), :] = jnp.concatenate(
            [tx.astype(jnp.bfloat16),
             jnp.ones((_BJP, _HE - _H), jnp.bfloat16)], axis=1)
        res = jax.lax.dot_general(
            x, ws_ref[...], (((1,), (0,)), ((), ())),
            preferred_element_type=jnp.float32,
            precision=_PHIGH) + bsr_ref[...]
        res_s[pl.ds(j * _BJP, _BJP), :] = res.astype(jnp.bfloat16)

    is_agg = ((j >= _J_AGG0) & (j < _J_EPI0)) | ((j >= _J_AGG1) &
                                                 (j < _J_EPI1))

    @pl.when(is_agg)
    def _():
        jj = j - jnp.where(j < _J_EPI0, _J_AGG0, _J_AGG1)
        base = jj * _BJA
        thr = (1e-6 * (inc_s[...] + 1e-8)).astype(jnp.bfloat16)
        m = mbf_s[pl.ds(base, _BJA), :]
        mm = jnp.where(m > thr, m, jnp.zeros_like(m))
        txe = txt_s[pl.ds(base, _BJA), :]
        part = jax.lax.dot_general(
            txe, mm, (((0,), (0,)), ((), ())),
            preferred_element_type=jnp.float32)

        first = (j == _J_AGG0) | (j == _J_AGG1)

        @pl.when(first)
        def _():
            acc_s[...] = part

        @pl.when(jnp.logical_not(first))
        def _():
            acc_s[...] += part

    def _epi_chunk(e, w2_ref, b2c_ref, gc_ref, btc_ref):
        lo = e * _EC
        sl = slice(lo, lo + _EC)
        s_row = acc_s[_H:_H + 1, sl]
        denom = s_row + 1e-8 * (inc_s[0:1, sl] + 1e-8)
        tfall = jnp.transpose(
            txt_s[sl, 0:_H]).astype(jnp.float32)
        aggt = jnp.where(s_row > 0.0, acc_s[0:_H, sl] / denom, tfall)
        outt = jax.lax.dot_general(
            w2_ref[...], aggt.astype(jnp.bfloat16), (((0,), (0,)), ((), ())),
            preferred_element_type=jnp.float32) + b2c_ref[...]
        return outt

    def _layernorm(outt, gc_ref, btc_ref):
        mu = jnp.mean(outt, axis=0, keepdims=True)
        var = jnp.mean((outt - mu) ** 2, axis=0, keepdims=True)
        return ((outt - mu) * jax.lax.rsqrt(var + 1e-5) * gc_ref[...]
                + btc_ref[...])

    @pl.when(j == _J_EPI0)
    def _():
        for e in range(_N // _EC):
            lo = e * _EC
            sl = slice(lo, lo + _EC)
            outt = _epi_chunk(e, w20_ref, b20c_ref, g0c_ref, bt0c_ref)
            outt = outt + jnp.transpose(res_s[sl, :]).astype(jnp.float32)
            outt = _layernorm(outt, g0c_ref, bt0c_ref)
            ht_s[:, sl] = outt.astype(jnp.bfloat16)
            tx1 = jax.lax.dot_general(
                outt.astype(jnp.bfloat16), w11_ref[...],
                (((0,), (0,)), ((), ())),
                preferred_element_type=jnp.float32) + b11r_ref[...]
            txt_s[sl, :] = jnp.concatenate(
                [tx1.astype(jnp.bfloat16),
                 jnp.ones((_EC, _HE - _H), jnp.bfloat16)], axis=1)

    @pl.when(j == _J_EPI1)
    def _():
        for e in range(_N // _EC):
            lo = e * _EC
            sl = slice(lo, lo + _EC)
            outt = _epi_chunk(e, w21_ref, b21c_ref, g1c_ref, bt1c_ref)
            outt = outt + ht_s[:, sl].astype(jnp.float32)
            outt = _layernorm(outt, g1c_ref, bt1c_ref)
            outt = jnp.maximum(outt, 0.0)
            out_ref[sl, :] = outt.T


def kernel(node_features, mobility_matrix, W1_0, b1_0, W2_0, b2_0, Ws_0,
           bs_0, g_0, bt_0, W1_1, b1_1, W2_1, b2_1, g_1, bt_1):
    col = lambda v: v.reshape(-1, 1)
    row = lambda v: v.reshape(1, -1)
    bf = lambda v: v.astype(jnp.bfloat16)
    const = lambda j: (0, 0)

    in_specs = [
        pl.BlockSpec((_BJP, _N), lambda j: (jnp.minimum(j, _NJP - 1), 0)),
        pl.BlockSpec((_BJP, 128), lambda j: (jnp.minimum(j, _NJP - 1), 0)),
        pl.BlockSpec((128, _H), const),
        pl.BlockSpec((1, _H), const),
        pl.BlockSpec((128, _H), const),
        pl.BlockSpec((1, _H), const),
        pl.BlockSpec((_H, _H), const),
        pl.BlockSpec((_H, 1), const),
        pl.BlockSpec((_H, 1), const),
        pl.BlockSpec((_H, 1), const),
        pl.BlockSpec((_H, _H), const),
        pl.BlockSpec((1, _H), const),
        pl.BlockSpec((_H, _H), const),
        pl.BlockSpec((_H, 1), const),
        pl.BlockSpec((_H, 1), const),
        pl.BlockSpec((_H, 1), const),
    ]
    inputs = [
        mobility_matrix, node_features,
        W1_0, row(b1_0), Ws_0, row(bs_0),
        bf(W2_0), col(b2_0), col(g_0), col(bt_0),
        bf(W1_1), row(b1_1), bf(W2_1), col(b2_1), col(g_1), col(bt_1),
    ]
    scratch = [
        pltpu.VMEM((_N, _N), jnp.bfloat16),
        pltpu.VMEM((_N, _HE), jnp.bfloat16),
        pltpu.VMEM((_N, _H), jnp.bfloat16),
        pltpu.VMEM((_H, _N), jnp.bfloat16),
        pltpu.VMEM((_HE, _N), jnp.float32),
        pltpu.VMEM((1, _N), jnp.float32),
    ]
    return pl.pallas_call(
        _body,
        grid=(_J_EPI1 + 1,),
        in_specs=in_specs,
        out_specs=pl.BlockSpec((_N, _H), lambda j: (0, 0)),
        out_shape=jax.ShapeDtypeStruct((_N, _H), jnp.float32),
        scratch_shapes=scratch,
        compiler_params=pltpu.CompilerParams(
            dimension_semantics=("arbitrary",),
        ),
    )(*inputs)
